# Initial kernel scaffold; baseline (speedup 1.0000x reference)
import jax
import jax.numpy as jnp
from jax.experimental import pallas as pl


def kernel(edge_index_list, node_feats_list, edge_feats_list, nodes_mask_list, params):
    raise NotImplementedError("write your pallas kernel here")


# speedup vs baseline: 11.4221x; 11.4221x over previous
import functools

import jax
import jax.numpy as jnp
from jax import lax
from jax.experimental import pallas as pl
from jax.experimental.pallas import tpu as pltpu
from jax.experimental.pallas import tpu_sc as plsc

N = 10000
E = 320000
F = 128
NC = 2
NS = 16
B = 80
ROWS2D = E // B
EPT = E // (NC * NS)
CPT = EPT // B
GC = 25
NG = CPT // GC
GE = GC * B
DEG_RPT = ROWS2D // NS
NGD = DEG_RPT // GC
RPT = N // NS
WCH = 125
NWCH = RPT // WCH


def _rsqrt16(v):
    i = lax.bitcast_convert_type(v, jnp.int32)
    i = jnp.int32(0x5F3759DF) - (i >> 1)
    y = lax.bitcast_convert_type(i, jnp.float32)
    for _ in range(3):
        y = y * (1.5 - 0.5 * v * y * y)
    return y


def _zero_rows(buf, nrows):
    zv = jnp.zeros((16,), jnp.float32)
    def body(r, _):
        for f in range(F // 16):
            buf[r, pl.ds(f * 16, 16)] = zv
        return 0
    lax.fori_loop(0, nrows, body, 0)


def _zero_acc_slice(acc, wo, sid):
    for q in range(NWCH):
        pltpu.sync_copy(wo, acc.at[pl.ds(sid * RPT + q * WCH, WCH)])


def _prop_group(table_h, srcv, dstv, normv, wo, acc):
    def chunk(c, _):
        pltpu.sync_copy(table_h.at[srcv.at[c]], wo.at[pl.ds(0, B)])
        def scale(r, _):
            bidx = jnp.full((16,), c * B + r, jnp.int32)
            bs = plsc.load_gather(normv, [bidx])
            for f in range(F // 16):
                wo[r, pl.ds(f * 16, 16)] = wo[r, pl.ds(f * 16, 16)] * bs
            return 0
        lax.fori_loop(0, B, scale, 0)
        pltpu.sync_copy(wo.at[pl.ds(0, B)], acc.at[dstv.at[c]], add=True)
        return 0
    lax.fori_loop(0, GC, chunk, 0)


def _writeout(acc, wo, out_h, cid, sid):
    for q in range(NWCH):
        r0 = sid * RPT + q * WCH
        pltpu.sync_copy(acc.at[pl.ds(r0, WCH)], wo)
        pltpu.sync_copy(wo, out_h.at[cid, pl.ds(r0, WCH)])


def _sc_pass1_body(src2_h, dst2_h, w2_h, x_h, tx1p_h, norm_h,
                   srcv, dstv, wv, normv, dis, wo, deg_sh, acc):
    cid = lax.axis_index("c")
    sid = lax.axis_index("s")
    wid = cid * NS + sid

    zv = jnp.zeros((16,), jnp.float32)
    def z16(i, _):
        dis[pl.ds(i * 16, 16)] = zv
        return 0
    lax.fori_loop(0, N // 16, z16, 0)
    pltpu.sync_copy(dis.at[pl.ds(sid * 624, 624)],
                    deg_sh.at[pl.ds(sid * 624, 624)])
    @pl.when(sid == 0)
    def _():
        pltpu.sync_copy(dis.at[pl.ds(NS * 624, N - NS * 624)],
                        deg_sh.at[pl.ds(NS * 624, N - NS * 624)])
    plsc.subcore_barrier()

    def dgroup(g, _):
        r0 = sid * DEG_RPT + g * GC
        pltpu.sync_copy(src2_h.at[pl.ds(r0, GC)], srcv)
        pltpu.sync_copy(w2_h.at[pl.ds(r0, GC)], wv)
        def dadd(c, _):
            pltpu.sync_copy(wv.at[c], deg_sh.at[srcv.at[c]], add=True)
            return 0
        lax.fori_loop(0, GC, dadd, 0)
        return 0
    lax.fori_loop(0, NGD, dgroup, 0)
    plsc.subcore_barrier()

    pltpu.sync_copy(deg_sh, dis)
    def dcomp(i, _):
        v = dis[pl.ds(i * 16, 16)]
        ok = v > 0.0
        y = _rsqrt16(jnp.where(ok, v, 1.0))
        dis[pl.ds(i * 16, 16)] = jnp.where(ok, y, 0.0)
        return 0
    lax.fori_loop(0, N // 16, dcomp, 0)

    _zero_rows(wo, WCH)
    _zero_acc_slice(acc, wo, sid)
    plsc.subcore_barrier()

    def group(g, _):
        r0 = wid * CPT + g * GC
        pltpu.sync_copy(src2_h.at[pl.ds(r0, GC)], srcv)
        pltpu.sync_copy(dst2_h.at[pl.ds(r0, GC)], dstv)
        pltpu.sync_copy(w2_h.at[pl.ds(r0, GC)], wv)
        def ncomp(j, _):
            for k in range(B // 16):
                s = srcv[j, pl.ds(k * 16, 16)]
                d = dstv[j, pl.ds(k * 16, 16)]
                wq = wv[j, pl.ds(k * 16, 16)]
                a = plsc.load_gather(dis, [s])
                b = plsc.load_gather(dis, [d])
                normv[pl.ds(j * B + k * 16, 16)] = -(a * wq * b)
            return 0
        lax.fori_loop(0, GC, ncomp, 0)
        pltpu.sync_copy(normv, norm_h.at[wid, pl.ds(g * GE, GE)])
        _prop_group(x_h, srcv, dstv, normv, wo, acc)
        return 0
    lax.fori_loop(0, NG, group, 0)
    plsc.subcore_barrier()
    _writeout(acc, wo, tx1p_h, cid, sid)


def _sc_pass2_body(src2_h, dst2_h, norm_h, tx1_h, tx2p_h,
                   srcv, dstv, normv, wo, acc):
    cid = lax.axis_index("c")
    sid = lax.axis_index("s")
    wid = cid * NS + sid

    _zero_rows(wo, WCH)
    _zero_acc_slice(acc, wo, sid)
    plsc.subcore_barrier()

    def group(g, _):
        r0 = wid * CPT + g * GC
        pltpu.sync_copy(src2_h.at[pl.ds(r0, GC)], srcv)
        pltpu.sync_copy(dst2_h.at[pl.ds(r0, GC)], dstv)
        pltpu.sync_copy(norm_h.at[wid, pl.ds(g * GE, GE)], normv)
        _prop_group(tx1_h, srcv, dstv, normv, wo, acc)
        return 0
    lax.fori_loop(0, NG, group, 0)
    plsc.subcore_barrier()
    _writeout(acc, wo, tx2p_h, cid, sid)


@functools.lru_cache(maxsize=1)
def _sc_kernels():
    mesh = plsc.VectorSubcoreMesh(
        core_axis_name="c", subcore_axis_name="s",
        num_cores=NC, num_subcores=NS)
    cp = pltpu.CompilerParams(use_tc_tiling_on_sc=False,
                              needs_layout_passes=False)
    sc1 = pl.kernel(
        _sc_pass1_body,
        out_type=(jax.ShapeDtypeStruct((NC, N, F), jnp.float32),
                  jax.ShapeDtypeStruct((NC * NS, EPT), jnp.float32)),
        mesh=mesh,
        compiler_params=cp,
        scratch_types=[
            pltpu.VMEM((GC, B), jnp.int32),
            pltpu.VMEM((GC, B), jnp.int32),
            pltpu.VMEM((GC, B), jnp.float32),
            pltpu.VMEM((GE,), jnp.float32),
            pltpu.VMEM((N,), jnp.float32),
            pltpu.VMEM((WCH, F), jnp.float32),
            pltpu.VMEM_SHARED((N,), jnp.float32),
            pltpu.VMEM_SHARED((N, F), jnp.float32),
        ],
    )
    sc2 = pl.kernel(
        _sc_pass2_body,
        out_type=jax.ShapeDtypeStruct((NC, N, F), jnp.float32),
        mesh=mesh,
        compiler_params=cp,
        scratch_types=[
            pltpu.VMEM((GC, B), jnp.int32),
            pltpu.VMEM((GC, B), jnp.int32),
            pltpu.VMEM((GE,), jnp.float32),
            pltpu.VMEM((WCH, F), jnp.float32),
            pltpu.VMEM_SHARED((N, F), jnp.float32),
        ],
    )
    return sc1, sc2


def _combine_body(a_ref, b_ref, o_ref):
    o_ref[...] = a_ref[...] + b_ref[...]


def _dense_body(x_ref, t1_ref, t2a_ref, t2b_ref, w_ref, bias_ref, wco_ref,
                o_ref):
    x = x_ref[...]
    t1 = t1_ref[...]
    t2 = 2.0 * (t2a_ref[...] + t2b_ref[...]) - x
    w = w_ref[...]
    g = (jnp.dot(x, w[0:F, :], preferred_element_type=jnp.float32)
         + jnp.dot(t1, w[F:2 * F, :], preferred_element_type=jnp.float32)
         + jnp.dot(t2, w[2 * F:3 * F, :], preferred_element_type=jnp.float32)
         + bias_ref[...])
    i_g = jax.nn.sigmoid(g[:, 0:F])
    t_g = jnp.tanh(g[:, F:2 * F])
    c = i_g * t_g
    o_g = jax.nn.sigmoid(g[:, 2 * F:3 * F] + wco_ref[...] * c)
    h = o_g * jnp.tanh(c)
    o_ref[...] = jnp.maximum(h, 0.0)


_RB = 1000
_GRID = N // _RB

_combine = pl.pallas_call(
    _combine_body,
    grid=(_GRID,),
    in_specs=[pl.BlockSpec((_RB, F), lambda i: (i, 0))] * 2,
    out_specs=pl.BlockSpec((_RB, F), lambda i: (i, 0)),
    out_shape=jax.ShapeDtypeStruct((N, F), jnp.float32),
)

_dense = pl.pallas_call(
    _dense_body,
    grid=(_GRID,),
    in_specs=[
        pl.BlockSpec((_RB, F), lambda i: (i, 0)),
        pl.BlockSpec((_RB, F), lambda i: (i, 0)),
        pl.BlockSpec((_RB, F), lambda i: (i, 0)),
        pl.BlockSpec((_RB, F), lambda i: (i, 0)),
        pl.BlockSpec((3 * F, 3 * F), lambda i: (0, 0)),
        pl.BlockSpec((1, 3 * F), lambda i: (0, 0)),
        pl.BlockSpec((1, F), lambda i: (0, 0)),
    ],
    out_specs=pl.BlockSpec((_RB, F), lambda i: (i, 0)),
    out_shape=jax.ShapeDtypeStruct((N, F), jnp.float32),
)


def kernel(edge_index_list, node_feats_list, edge_feats_list,
           nodes_mask_list, params):
    ei = edge_index_list[-1].astype(jnp.int32)
    src2 = ei[0].reshape(ROWS2D, B)
    dst2 = ei[1].reshape(ROWS2D, B)
    w2 = edge_feats_list[-1].astype(jnp.float32).reshape(ROWS2D, B)
    x = node_feats_list[-1].astype(jnp.float32)

    sc1, sc2 = _sc_kernels()
    tx1p, norm2 = sc1(src2, dst2, w2, x)
    tx1 = _combine(tx1p[0], tx1p[1])
    tx2p = sc2(src2, dst2, norm2, tx1)

    gates = "ico"
    wcat = jnp.concatenate(
        [jnp.concatenate([params["W_x_" + g][k] for g in gates], axis=1)
         for k in range(3)], axis=0)
    bias = jnp.concatenate(
        [params["b_x_" + g] + params["b_h_" + g] + params["b_" + g][0]
         for g in gates])[None, :]
    return _dense(x, tx1, tx2p[0], tx2p[1], wcat, bias, params["w_c_o"])


# speedup vs baseline: 16.1996x; 1.4183x over previous
import functools

import jax
import jax.numpy as jnp
from jax import lax
from jax.experimental import pallas as pl
from jax.experimental.pallas import tpu as pltpu
from jax.experimental.pallas import tpu_sc as plsc

N = 10000
E = 320000
F = 128
NC = 2
NS = 16
B = 80
ROWS2D = E // B
EPT = E // (NC * NS)
CPT = EPT // B
GC = 25
NG = CPT // GC
GE = GC * B
DB = 125
DROWS = E // DB
DEG_RPT = DROWS // NS
DGC = 8
NDG = DEG_RPT // DGC
RPT = N // NS


def _rsqrt16(v):
    i = lax.bitcast_convert_type(v, jnp.int32)
    i = jnp.int32(0x5F3759DF) - (i >> 1)
    y = lax.bitcast_convert_type(i, jnp.float32)
    for _ in range(3):
        y = y * (1.5 - 0.5 * v * y * y)
    return y


def _zero_rows(buf, nrows):
    zv = jnp.zeros((16,), jnp.float32)
    def body(r, _):
        for f in range(F // 16):
            buf[r, pl.ds(f * 16, 16)] = zv
        return 0
    lax.fori_loop(0, nrows, body, 0)


def _zero_acc_slice(acc, row, sid):
    base = sid * RPT
    for q in range(RPT // B):
        pltpu.sync_copy(row, acc.at[pl.ds(base + q * B, B)])
    rem = RPT % B
    pltpu.sync_copy(row.at[pl.ds(0, rem)],
                    acc.at[pl.ds(base + (RPT // B) * B, rem)])


def _scale_rows(row, normv, c):
    def scale(r, _):
        bidx = jnp.full((16,), c * B + r, jnp.int32)
        bs = plsc.load_gather(normv, [bidx])
        for f in range(F // 16):
            row[r, pl.ds(f * 16, 16)] = row[r, pl.ds(f * 16, 16)] * bs
        return 0
    lax.fori_loop(0, B, scale, 0)


def _prop_group(table_h, srcv, dstv, normv, rowA, rowB, semA, semB, acc):
    pltpu.async_copy(table_h.at[srcv.at[0]], rowA, semA)
    def pair(i, _):
        c0 = 2 * i
        c1 = c0 + 1
        pltpu.make_async_copy(table_h.at[srcv.at[c0]], rowA, semA).wait()
        pltpu.async_copy(table_h.at[srcv.at[c1]], rowB, semB)
        _scale_rows(rowA, normv, c0)
        pltpu.sync_copy(rowA, acc.at[dstv.at[c0]], add=True)
        pltpu.make_async_copy(table_h.at[srcv.at[c1]], rowB, semB).wait()
        @pl.when(i < GC // 2 - 1)
        def _():
            pltpu.async_copy(table_h.at[srcv.at[c0 + 2]], rowA, semA)
        _scale_rows(rowB, normv, c1)
        pltpu.sync_copy(rowB, acc.at[dstv.at[c1]], add=True)
        return 0
    lax.fori_loop(0, GC // 2, pair, 0)
    c = GC - 1
    pltpu.sync_copy(table_h.at[srcv.at[c]], rowA)
    _scale_rows(rowA, normv, c)
    pltpu.sync_copy(rowA, acc.at[dstv.at[c]], add=True)


def _writeout(acc, row, out_h, cid, sid):
    base = sid * RPT
    for q in range(RPT // B):
        r0 = base + q * B
        pltpu.sync_copy(acc.at[pl.ds(r0, B)], row)
        pltpu.sync_copy(row, out_h.at[cid, pl.ds(r0, B)])
    rem = RPT % B
    r0 = base + (RPT // B) * B
    pltpu.sync_copy(acc.at[pl.ds(r0, rem)], row.at[pl.ds(0, rem)])
    pltpu.sync_copy(row.at[pl.ds(0, rem)], out_h.at[cid, pl.ds(r0, rem)])


def _sc_pass1_body(src2_h, dst2_h, w2_h, srcd_h, wd_h, x_h, tx1p_h, norm_h,
                   srcv, dstv, wv, normv, dis, rowA, rowB, degi, degw,
                   semA, semB, deg_sh, acc):
    cid = lax.axis_index("c")
    sid = lax.axis_index("s")
    wid = cid * NS + sid

    zv = jnp.zeros((16,), jnp.float32)
    def z16(i, _):
        dis[pl.ds(i * 16, 16)] = zv
        return 0
    lax.fori_loop(0, N // 16, z16, 0)
    pltpu.sync_copy(dis.at[pl.ds(sid * 624, 624)],
                    deg_sh.at[pl.ds(sid * 624, 624)])
    @pl.when(sid == 0)
    def _():
        pltpu.sync_copy(dis.at[pl.ds(NS * 624, N - NS * 624)],
                        deg_sh.at[pl.ds(NS * 624, N - NS * 624)])
    plsc.subcore_barrier()

    def dgroup(g, _):
        r0 = sid * DEG_RPT + g * DGC
        pltpu.sync_copy(srcd_h.at[pl.ds(r0, DGC)], degi)
        pltpu.sync_copy(wd_h.at[pl.ds(r0, DGC)], degw)
        def dadd(c, _):
            pltpu.sync_copy(degw.at[c], deg_sh.at[degi.at[c]], add=True)
            return 0
        lax.fori_loop(0, DGC, dadd, 0)
        return 0
    lax.fori_loop(0, NDG, dgroup, 0)
    plsc.subcore_barrier()

    pltpu.sync_copy(deg_sh, dis)
    def dcomp(i, _):
        v = dis[pl.ds(i * 16, 16)]
        ok = v > 0.0
        y = _rsqrt16(jnp.where(ok, v, 1.0))
        dis[pl.ds(i * 16, 16)] = jnp.where(ok, y, 0.0)
        return 0
    lax.fori_loop(0, N // 16, dcomp, 0)

    _zero_rows(rowA, B)
    _zero_acc_slice(acc, rowA, sid)
    plsc.subcore_barrier()

    def group(g, _):
        r0 = wid * CPT + g * GC
        pltpu.sync_copy(src2_h.at[pl.ds(r0, GC)], srcv)
        pltpu.sync_copy(dst2_h.at[pl.ds(r0, GC)], dstv)
        pltpu.sync_copy(w2_h.at[pl.ds(r0, GC)], wv)
        def ncomp(j, _):
            for k in range(B // 16):
                s = srcv[j, pl.ds(k * 16, 16)]
                d = dstv[j, pl.ds(k * 16, 16)]
                wq = wv[j, pl.ds(k * 16, 16)]
                a = plsc.load_gather(dis, [s])
                b = plsc.load_gather(dis, [d])
                normv[pl.ds(j * B + k * 16, 16)] = -(a * wq * b)
            return 0
        lax.fori_loop(0, GC, ncomp, 0)
        pltpu.sync_copy(normv, norm_h.at[wid, pl.ds(g * GE, GE)])
        _prop_group(x_h, srcv, dstv, normv, rowA, rowB, semA, semB, acc)
        return 0
    lax.fori_loop(0, NG, group, 0)
    plsc.subcore_barrier()
    _writeout(acc, rowA, tx1p_h, cid, sid)


def _sc_pass2_body(src2_h, dst2_h, norm_h, tx1_h, tx2p_h,
                   srcv, dstv, normv, rowA, rowB, semA, semB, acc):
    cid = lax.axis_index("c")
    sid = lax.axis_index("s")
    wid = cid * NS + sid

    _zero_rows(rowA, B)
    _zero_acc_slice(acc, rowA, sid)
    plsc.subcore_barrier()

    def group(g, _):
        r0 = wid * CPT + g * GC
        pltpu.sync_copy(src2_h.at[pl.ds(r0, GC)], srcv)
        pltpu.sync_copy(dst2_h.at[pl.ds(r0, GC)], dstv)
        pltpu.sync_copy(norm_h.at[wid, pl.ds(g * GE, GE)], normv)
        _prop_group(tx1_h, srcv, dstv, normv, rowA, rowB, semA, semB, acc)
        return 0
    lax.fori_loop(0, NG, group, 0)
    plsc.subcore_barrier()
    _writeout(acc, rowA, tx2p_h, cid, sid)


@functools.lru_cache(maxsize=1)
def _sc_kernels():
    mesh = plsc.VectorSubcoreMesh(
        core_axis_name="c", subcore_axis_name="s",
        num_cores=NC, num_subcores=NS)
    cp = pltpu.CompilerParams(use_tc_tiling_on_sc=False,
                              needs_layout_passes=False)
    sc1 = pl.kernel(
        _sc_pass1_body,
        out_type=(jax.ShapeDtypeStruct((NC, N, F), jnp.float32),
                  jax.ShapeDtypeStruct((NC * NS, EPT), jnp.float32)),
        mesh=mesh,
        compiler_params=cp,
        scratch_types=[
            pltpu.VMEM((GC, B), jnp.int32),
            pltpu.VMEM((GC, B), jnp.int32),
            pltpu.VMEM((GC, B), jnp.float32),
            pltpu.VMEM((GE,), jnp.float32),
            pltpu.VMEM((N,), jnp.float32),
            pltpu.VMEM((B, F), jnp.float32),
            pltpu.VMEM((B, F), jnp.float32),
            pltpu.VMEM((DGC, DB), jnp.int32),
            pltpu.VMEM((DGC, DB), jnp.float32),
            pltpu.SemaphoreType.DMA,
            pltpu.SemaphoreType.DMA,
            pltpu.VMEM_SHARED((N,), jnp.float32),
            pltpu.VMEM_SHARED((N, F), jnp.float32),
        ],
    )
    sc2 = pl.kernel(
        _sc_pass2_body,
        out_type=jax.ShapeDtypeStruct((NC, N, F), jnp.float32),
        mesh=mesh,
        compiler_params=cp,
        scratch_types=[
            pltpu.VMEM((GC, B), jnp.int32),
            pltpu.VMEM((GC, B), jnp.int32),
            pltpu.VMEM((GE,), jnp.float32),
            pltpu.VMEM((B, F), jnp.float32),
            pltpu.VMEM((B, F), jnp.float32),
            pltpu.SemaphoreType.DMA,
            pltpu.SemaphoreType.DMA,
            pltpu.VMEM_SHARED((N, F), jnp.float32),
        ],
    )
    return sc1, sc2


def _combine_body(a_ref, b_ref, o_ref):
    o_ref[...] = a_ref[...] + b_ref[...]


def _dense_body(x_ref, t1_ref, t2a_ref, t2b_ref, w_ref, bias_ref, wco_ref,
                o_ref):
    x = x_ref[...]
    t1 = t1_ref[...]
    t2 = 2.0 * (t2a_ref[...] + t2b_ref[...]) - x
    w = w_ref[...]
    g = (jnp.dot(x, w[0:F, :], preferred_element_type=jnp.float32)
         + jnp.dot(t1, w[F:2 * F, :], preferred_element_type=jnp.float32)
         + jnp.dot(t2, w[2 * F:3 * F, :], preferred_element_type=jnp.float32)
         + bias_ref[...])
    i_g = jax.nn.sigmoid(g[:, 0:F])
    t_g = jnp.tanh(g[:, F:2 * F])
    c = i_g * t_g
    o_g = jax.nn.sigmoid(g[:, 2 * F:3 * F] + wco_ref[...] * c)
    h = o_g * jnp.tanh(c)
    o_ref[...] = jnp.maximum(h, 0.0)


_RB = 1000
_GRID = N // _RB

_combine = pl.pallas_call(
    _combine_body,
    grid=(_GRID,),
    in_specs=[pl.BlockSpec((_RB, F), lambda i: (i, 0))] * 2,
    out_specs=pl.BlockSpec((_RB, F), lambda i: (i, 0)),
    out_shape=jax.ShapeDtypeStruct((N, F), jnp.float32),
)

_dense = pl.pallas_call(
    _dense_body,
    grid=(_GRID,),
    in_specs=[
        pl.BlockSpec((_RB, F), lambda i: (i, 0)),
        pl.BlockSpec((_RB, F), lambda i: (i, 0)),
        pl.BlockSpec((_RB, F), lambda i: (i, 0)),
        pl.BlockSpec((_RB, F), lambda i: (i, 0)),
        pl.BlockSpec((3 * F, 3 * F), lambda i: (0, 0)),
        pl.BlockSpec((1, 3 * F), lambda i: (0, 0)),
        pl.BlockSpec((1, F), lambda i: (0, 0)),
    ],
    out_specs=pl.BlockSpec((_RB, F), lambda i: (i, 0)),
    out_shape=jax.ShapeDtypeStruct((N, F), jnp.float32),
)


def kernel(edge_index_list, node_feats_list, edge_feats_list,
           nodes_mask_list, params):
    ei = edge_index_list[-1].astype(jnp.int32)
    src2 = ei[0].reshape(ROWS2D, B)
    dst2 = ei[1].reshape(ROWS2D, B)
    w = edge_feats_list[-1].astype(jnp.float32)
    w2 = w.reshape(ROWS2D, B)
    srcd = ei[0].reshape(DROWS, DB)
    wd = w.reshape(DROWS, DB)
    x = node_feats_list[-1].astype(jnp.float32)

    sc1, sc2 = _sc_kernels()
    tx1p, norm2 = sc1(src2, dst2, w2, srcd, wd, x)
    tx1 = _combine(tx1p[0], tx1p[1])
    tx2p = sc2(src2, dst2, norm2, tx1)

    gates = "ico"
    wcat = jnp.concatenate(
        [jnp.concatenate([params["W_x_" + g][k] for g in gates], axis=1)
         for k in range(3)], axis=0)
    bias = jnp.concatenate(
        [params["b_x_" + g] + params["b_h_" + g] + params["b_" + g][0]
         for g in gates])[None, :]
    return _dense(x, tx1, tx2p[0], tx2p[1], wcat, bias, params["w_c_o"])


# speedup vs baseline: 16.5436x; 1.0212x over previous
import functools

import jax
import jax.numpy as jnp
from jax import lax
from jax.experimental import pallas as pl
from jax.experimental.pallas import tpu as pltpu
from jax.experimental.pallas import tpu_sc as plsc

N = 10000
E = 320000
F = 128
NC = 2
NS = 16
B = 80
ROWS2D = E // B
EPT = E // (NC * NS)
CPT = EPT // B
GC = 25
NG = CPT // GC
GE = GC * B
DB = 125
DROWS = E // DB
DEG_RPT = DROWS // NS
DGC = 8
NDG = DEG_RPT // DGC
RPT = N // NS


def _rsqrt16(v):
    i = lax.bitcast_convert_type(v, jnp.int32)
    i = jnp.int32(0x5F3759DF) - (i >> 1)
    y = lax.bitcast_convert_type(i, jnp.float32)
    for _ in range(3):
        y = y * (1.5 - 0.5 * v * y * y)
    return y


def _zero_rows(buf, nrows):
    zv = jnp.zeros((16,), jnp.float32)
    def body(r, _):
        for f in range(F // 16):
            buf[r, pl.ds(f * 16, 16)] = zv
        return 0
    lax.fori_loop(0, nrows, body, 0)


def _zero_acc_slice(acc, row, sid):
    base = sid * RPT
    for q in range(RPT // B):
        pltpu.sync_copy(row, acc.at[pl.ds(base + q * B, B)])
    rem = RPT % B
    pltpu.sync_copy(row.at[pl.ds(0, rem)],
                    acc.at[pl.ds(base + (RPT // B) * B, rem)])


def _scale_rows(row, normv, c):
    def scale(r, _):
        bidx = jnp.full((16,), c * B + r, jnp.int32)
        bs = plsc.load_gather(normv, [bidx])
        for f in range(F // 16):
            row[r, pl.ds(f * 16, 16)] = row[r, pl.ds(f * 16, 16)] * bs
        return 0
    lax.fori_loop(0, B, scale, 0)


def _prop_group(table_h, srcv, dstv, normv, rowA, rowB,
                semA, semB, ssemA, ssemB, acc):
    pltpu.async_copy(table_h.at[srcv.at[0]], rowA, semA)
    def pair(i, _):
        c0 = 2 * i
        c1 = c0 + 1
        @pl.when(i > 0)
        def _():
            pltpu.make_async_copy(
                rowB, acc.at[dstv.at[c1 - 2]], ssemB).wait()
        pltpu.async_copy(table_h.at[srcv.at[c1]], rowB, semB)
        pltpu.make_async_copy(table_h.at[srcv.at[c0]], rowA, semA).wait()
        _scale_rows(rowA, normv, c0)
        pltpu.async_copy(rowA, acc.at[dstv.at[c0]], ssemA, add=True)
        @pl.when(i < GC // 2 - 1)
        def _():
            pltpu.make_async_copy(rowA, acc.at[dstv.at[c0]], ssemA).wait()
            pltpu.async_copy(table_h.at[srcv.at[c0 + 2]], rowA, semA)
        pltpu.make_async_copy(table_h.at[srcv.at[c1]], rowB, semB).wait()
        _scale_rows(rowB, normv, c1)
        pltpu.async_copy(rowB, acc.at[dstv.at[c1]], ssemB, add=True)
        return 0
    lax.fori_loop(0, GC // 2, pair, 0)
    c = GC - 1
    pltpu.make_async_copy(rowA, acc.at[dstv.at[c - 2]], ssemA).wait()
    pltpu.sync_copy(table_h.at[srcv.at[c]], rowA)
    _scale_rows(rowA, normv, c)
    pltpu.make_async_copy(rowB, acc.at[dstv.at[c - 1]], ssemB).wait()
    pltpu.sync_copy(rowA, acc.at[dstv.at[c]], add=True)


def _writeout(acc, row, out_h, cid, sid):
    base = sid * RPT
    for q in range(RPT // B):
        r0 = base + q * B
        pltpu.sync_copy(acc.at[pl.ds(r0, B)], row)
        pltpu.sync_copy(row, out_h.at[cid, pl.ds(r0, B)])
    rem = RPT % B
    r0 = base + (RPT // B) * B
    pltpu.sync_copy(acc.at[pl.ds(r0, rem)], row.at[pl.ds(0, rem)])
    pltpu.sync_copy(row.at[pl.ds(0, rem)], out_h.at[cid, pl.ds(r0, rem)])


def _sc_pass1_body(src2_h, dst2_h, w2_h, srcd_h, wd_h, x_h, tx1p_h, norm_h,
                   srcv, dstv, wv, normv, dis, rowA, rowB, degi, degw,
                   semA, semB, ssemA, ssemB, deg_sh, acc):
    cid = lax.axis_index("c")
    sid = lax.axis_index("s")
    wid = cid * NS + sid

    zv = jnp.zeros((16,), jnp.float32)
    def z16(i, _):
        dis[pl.ds(i * 16, 16)] = zv
        return 0
    lax.fori_loop(0, N // 16, z16, 0)
    pltpu.sync_copy(dis.at[pl.ds(sid * 624, 624)],
                    deg_sh.at[pl.ds(sid * 624, 624)])
    @pl.when(sid == 0)
    def _():
        pltpu.sync_copy(dis.at[pl.ds(NS * 624, N - NS * 624)],
                        deg_sh.at[pl.ds(NS * 624, N - NS * 624)])
    plsc.subcore_barrier()

    def dgroup(g, _):
        r0 = sid * DEG_RPT + g * DGC
        pltpu.sync_copy(srcd_h.at[pl.ds(r0, DGC)], degi)
        pltpu.sync_copy(wd_h.at[pl.ds(r0, DGC)], degw)
        for c in range(DGC):
            pltpu.async_copy(degw.at[c], deg_sh.at[degi.at[c]], semA,
                             add=True)
        for c in range(DGC):
            pltpu.make_async_copy(degw.at[c], deg_sh.at[degi.at[c]],
                                  semA).wait()
        return 0
    lax.fori_loop(0, NDG, dgroup, 0)
    plsc.subcore_barrier()

    pltpu.sync_copy(deg_sh, dis)
    def dcomp(i, _):
        v = dis[pl.ds(i * 16, 16)]
        ok = v > 0.0
        y = _rsqrt16(jnp.where(ok, v, 1.0))
        dis[pl.ds(i * 16, 16)] = jnp.where(ok, y, 0.0)
        return 0
    lax.fori_loop(0, N // 16, dcomp, 0)

    _zero_rows(rowA, B)
    _zero_acc_slice(acc, rowA, sid)
    plsc.subcore_barrier()

    def group(g, _):
        r0 = wid * CPT + g * GC
        pltpu.sync_copy(src2_h.at[pl.ds(r0, GC)], srcv)
        pltpu.sync_copy(dst2_h.at[pl.ds(r0, GC)], dstv)
        pltpu.sync_copy(w2_h.at[pl.ds(r0, GC)], wv)
        def ncomp(j, _):
            for k in range(B // 16):
                s = srcv[j, pl.ds(k * 16, 16)]
                d = dstv[j, pl.ds(k * 16, 16)]
                wq = wv[j, pl.ds(k * 16, 16)]
                a = plsc.load_gather(dis, [s])
                b = plsc.load_gather(dis, [d])
                normv[pl.ds(j * B + k * 16, 16)] = -(a * wq * b)
            return 0
        lax.fori_loop(0, GC, ncomp, 0)
        pltpu.sync_copy(normv, norm_h.at[wid, pl.ds(g * GE, GE)])
        _prop_group(x_h, srcv, dstv, normv, rowA, rowB,
                    semA, semB, ssemA, ssemB, acc)
        return 0
    lax.fori_loop(0, NG, group, 0)
    plsc.subcore_barrier()
    _writeout(acc, rowA, tx1p_h, cid, sid)


def _sc_pass2_body(src2_h, dst2_h, norm_h, tx1_h, tx2p_h,
                   srcv, dstv, normv, rowA, rowB,
                   semA, semB, ssemA, ssemB, acc):
    cid = lax.axis_index("c")
    sid = lax.axis_index("s")
    wid = cid * NS + sid

    _zero_rows(rowA, B)
    _zero_acc_slice(acc, rowA, sid)
    plsc.subcore_barrier()

    def group(g, _):
        r0 = wid * CPT + g * GC
        pltpu.sync_copy(src2_h.at[pl.ds(r0, GC)], srcv)
        pltpu.sync_copy(dst2_h.at[pl.ds(r0, GC)], dstv)
        pltpu.sync_copy(norm_h.at[wid, pl.ds(g * GE, GE)], normv)
        _prop_group(tx1_h, srcv, dstv, normv, rowA, rowB,
                    semA, semB, ssemA, ssemB, acc)
        return 0
    lax.fori_loop(0, NG, group, 0)
    plsc.subcore_barrier()
    _writeout(acc, rowA, tx2p_h, cid, sid)


@functools.lru_cache(maxsize=1)
def _sc_kernels():
    mesh = plsc.VectorSubcoreMesh(
        core_axis_name="c", subcore_axis_name="s",
        num_cores=NC, num_subcores=NS)
    cp = pltpu.CompilerParams(use_tc_tiling_on_sc=False,
                              needs_layout_passes=False)
    sc1 = pl.kernel(
        _sc_pass1_body,
        out_type=(jax.ShapeDtypeStruct((NC, N, F), jnp.float32),
                  jax.ShapeDtypeStruct((NC * NS, EPT), jnp.float32)),
        mesh=mesh,
        compiler_params=cp,
        scratch_types=[
            pltpu.VMEM((GC, B), jnp.int32),
            pltpu.VMEM((GC, B), jnp.int32),
            pltpu.VMEM((GC, B), jnp.float32),
            pltpu.VMEM((GE,), jnp.float32),
            pltpu.VMEM((N,), jnp.float32),
            pltpu.VMEM((B, F), jnp.float32),
            pltpu.VMEM((B, F), jnp.float32),
            pltpu.VMEM((DGC, DB), jnp.int32),
            pltpu.VMEM((DGC, DB), jnp.float32),
            pltpu.SemaphoreType.DMA,
            pltpu.SemaphoreType.DMA,
            pltpu.SemaphoreType.DMA,
            pltpu.SemaphoreType.DMA,
            pltpu.VMEM_SHARED((N,), jnp.float32),
            pltpu.VMEM_SHARED((N, F), jnp.float32),
        ],
    )
    sc2 = pl.kernel(
        _sc_pass2_body,
        out_type=jax.ShapeDtypeStruct((NC, N, F), jnp.float32),
        mesh=mesh,
        compiler_params=cp,
        scratch_types=[
            pltpu.VMEM((GC, B), jnp.int32),
            pltpu.VMEM((GC, B), jnp.int32),
            pltpu.VMEM((GE,), jnp.float32),
            pltpu.VMEM((B, F), jnp.float32),
            pltpu.VMEM((B, F), jnp.float32),
            pltpu.SemaphoreType.DMA,
            pltpu.SemaphoreType.DMA,
            pltpu.SemaphoreType.DMA,
            pltpu.SemaphoreType.DMA,
            pltpu.VMEM_SHARED((N, F), jnp.float32),
        ],
    )
    return sc1, sc2


def _combine_body(a_ref, b_ref, o_ref):
    o_ref[...] = a_ref[...] + b_ref[...]


def _dense_body(x_ref, t1_ref, t2a_ref, t2b_ref, w_ref, bias_ref, wco_ref,
                o_ref):
    x = x_ref[...]
    t1 = t1_ref[...]
    t2 = 2.0 * (t2a_ref[...] + t2b_ref[...]) - x
    w = w_ref[...]
    g = (jnp.dot(x, w[0:F, :], preferred_element_type=jnp.float32)
         + jnp.dot(t1, w[F:2 * F, :], preferred_element_type=jnp.float32)
         + jnp.dot(t2, w[2 * F:3 * F, :], preferred_element_type=jnp.float32)
         + bias_ref[...])
    i_g = jax.nn.sigmoid(g[:, 0:F])
    t_g = jnp.tanh(g[:, F:2 * F])
    c = i_g * t_g
    o_g = jax.nn.sigmoid(g[:, 2 * F:3 * F] + wco_ref[...] * c)
    h = o_g * jnp.tanh(c)
    o_ref[...] = jnp.maximum(h, 0.0)


_RB = 1000
_GRID = N // _RB

_combine = pl.pallas_call(
    _combine_body,
    grid=(_GRID,),
    in_specs=[pl.BlockSpec((_RB, F), lambda i: (i, 0))] * 2,
    out_specs=pl.BlockSpec((_RB, F), lambda i: (i, 0)),
    out_shape=jax.ShapeDtypeStruct((N, F), jnp.float32),
)

_dense = pl.pallas_call(
    _dense_body,
    grid=(_GRID,),
    in_specs=[
        pl.BlockSpec((_RB, F), lambda i: (i, 0)),
        pl.BlockSpec((_RB, F), lambda i: (i, 0)),
        pl.BlockSpec((_RB, F), lambda i: (i, 0)),
        pl.BlockSpec((_RB, F), lambda i: (i, 0)),
        pl.BlockSpec((3 * F, 3 * F), lambda i: (0, 0)),
        pl.BlockSpec((1, 3 * F), lambda i: (0, 0)),
        pl.BlockSpec((1, F), lambda i: (0, 0)),
    ],
    out_specs=pl.BlockSpec((_RB, F), lambda i: (i, 0)),
    out_shape=jax.ShapeDtypeStruct((N, F), jnp.float32),
)


def kernel(edge_index_list, node_feats_list, edge_feats_list,
           nodes_mask_list, params):
    ei = edge_index_list[-1].astype(jnp.int32)
    src2 = ei[0].reshape(ROWS2D, B)
    dst2 = ei[1].reshape(ROWS2D, B)
    w = edge_feats_list[-1].astype(jnp.float32)
    w2 = w.reshape(ROWS2D, B)
    srcd = ei[0].reshape(DROWS, DB)
    wd = w.reshape(DROWS, DB)
    x = node_feats_list[-1].astype(jnp.float32)

    sc1, sc2 = _sc_kernels()
    tx1p, norm2 = sc1(src2, dst2, w2, srcd, wd, x)
    tx1 = _combine(tx1p[0], tx1p[1])
    tx2p = sc2(src2, dst2, norm2, tx1)

    gates = "ico"
    wcat = jnp.concatenate(
        [jnp.concatenate([params["W_x_" + g][k] for g in gates], axis=1)
         for k in range(3)], axis=0)
    bias = jnp.concatenate(
        [params["b_x_" + g] + params["b_h_" + g] + params["b_" + g][0]
         for g in gates])[None, :]
    return _dense(x, tx1, tx2p[0], tx2p[1], wcat, bias, params["w_c_o"])


# speedup vs baseline: 19.3664x; 1.1706x over previous
import functools

import jax
import jax.numpy as jnp
from jax import lax
from jax.experimental import pallas as pl
from jax.experimental.pallas import tpu as pltpu
from jax.experimental.pallas import tpu_sc as plsc

N = 10000
E = 320000
F = 128
NC = 2
NS = 16
B = 80
ROWS2D = E // B
EPT = E // (NC * NS)
CPT = EPT // B
GC = 25
NG = CPT // GC
GE = GC * B
DB = 125
DROWS = E // DB
DEG_RPT = DROWS // NS
DGC = 8
NDG = DEG_RPT // DGC
RPT = N // NS


def _rsqrt16(v):
    i = lax.bitcast_convert_type(v, jnp.int32)
    i = jnp.int32(0x5F3759DF) - (i >> 1)
    y = lax.bitcast_convert_type(i, jnp.float32)
    for _ in range(3):
        y = y * (1.5 - 0.5 * v * y * y)
    return y


def _zero_rows(buf, nrows):
    zv = jnp.zeros((16,), jnp.float32)
    def body(r, _):
        for f in range(F // 16):
            buf[r, pl.ds(f * 16, 16)] = zv
        return 0
    lax.fori_loop(0, nrows, body, 0)


def _zero_acc_slice(acc, row, sid):
    base = sid * RPT
    for q in range(RPT // B):
        pltpu.sync_copy(row, acc.at[pl.ds(base + q * B, B)])
    rem = RPT % B
    pltpu.sync_copy(row.at[pl.ds(0, rem)],
                    acc.at[pl.ds(base + (RPT // B) * B, rem)])


def _scale_rows(row, normv, c):
    @plsc.parallel_loop(0, B, unroll=4)
    def _(r):
        bidx = jnp.full((16,), c * B + r, jnp.int32)
        bs = plsc.load_gather(normv, [bidx])
        for f in range(F // 16):
            row[r, pl.ds(f * 16, 16)] = row[r, pl.ds(f * 16, 16)] * bs


def _prop_group(table_h, srcv, dstv, normv, rowA, rowB,
                semA, semB, ssemA, ssemB, acc):
    pltpu.async_copy(table_h.at[srcv.at[0]], rowA, semA)
    def pair(i, _):
        c0 = 2 * i
        c1 = c0 + 1
        @pl.when(i > 0)
        def _():
            pltpu.make_async_copy(
                rowB, acc.at[dstv.at[c1 - 2]], ssemB).wait()
        pltpu.async_copy(table_h.at[srcv.at[c1]], rowB, semB)
        pltpu.make_async_copy(table_h.at[srcv.at[c0]], rowA, semA).wait()
        _scale_rows(rowA, normv, c0)
        pltpu.async_copy(rowA, acc.at[dstv.at[c0]], ssemA, add=True)
        @pl.when(i < GC // 2 - 1)
        def _():
            pltpu.make_async_copy(rowA, acc.at[dstv.at[c0]], ssemA).wait()
            pltpu.async_copy(table_h.at[srcv.at[c0 + 2]], rowA, semA)
        pltpu.make_async_copy(table_h.at[srcv.at[c1]], rowB, semB).wait()
        _scale_rows(rowB, normv, c1)
        pltpu.async_copy(rowB, acc.at[dstv.at[c1]], ssemB, add=True)
        return 0
    lax.fori_loop(0, GC // 2, pair, 0)
    c = GC - 1
    pltpu.make_async_copy(rowA, acc.at[dstv.at[c - 2]], ssemA).wait()
    pltpu.sync_copy(table_h.at[srcv.at[c]], rowA)
    _scale_rows(rowA, normv, c)
    pltpu.make_async_copy(rowB, acc.at[dstv.at[c - 1]], ssemB).wait()
    pltpu.sync_copy(rowA, acc.at[dstv.at[c]], add=True)


def _writeout(acc, row, out_h, cid, sid):
    base = sid * RPT
    for q in range(RPT // B):
        r0 = base + q * B
        pltpu.sync_copy(acc.at[pl.ds(r0, B)], row)
        pltpu.sync_copy(row, out_h.at[cid, pl.ds(r0, B)])
    rem = RPT % B
    r0 = base + (RPT // B) * B
    pltpu.sync_copy(acc.at[pl.ds(r0, rem)], row.at[pl.ds(0, rem)])
    pltpu.sync_copy(row.at[pl.ds(0, rem)], out_h.at[cid, pl.ds(r0, rem)])


def _sc_pass1_body(src2_h, dst2_h, w2_h, srcd_h, wd_h, x_h, tx1p_h, norm_h,
                   srcv, dstv, wv, normv, dis, rowA, rowB, degi, degw,
                   semA, semB, ssemA, ssemB, deg_sh, acc):
    cid = lax.axis_index("c")
    sid = lax.axis_index("s")
    wid = cid * NS + sid

    zv = jnp.zeros((16,), jnp.float32)
    def z16(i, _):
        dis[pl.ds(i * 16, 16)] = zv
        return 0
    lax.fori_loop(0, N // 16, z16, 0)
    pltpu.sync_copy(dis.at[pl.ds(sid * 624, 624)],
                    deg_sh.at[pl.ds(sid * 624, 624)])
    @pl.when(sid == 0)
    def _():
        pltpu.sync_copy(dis.at[pl.ds(NS * 624, N - NS * 624)],
                        deg_sh.at[pl.ds(NS * 624, N - NS * 624)])
    plsc.subcore_barrier()

    def dgroup(g, _):
        r0 = sid * DEG_RPT + g * DGC
        pltpu.sync_copy(srcd_h.at[pl.ds(r0, DGC)], degi)
        pltpu.sync_copy(wd_h.at[pl.ds(r0, DGC)], degw)
        for c in range(DGC):
            pltpu.async_copy(degw.at[c], deg_sh.at[degi.at[c]], semA,
                             add=True)
        for c in range(DGC):
            pltpu.make_async_copy(degw.at[c], deg_sh.at[degi.at[c]],
                                  semA).wait()
        return 0
    lax.fori_loop(0, NDG, dgroup, 0)
    plsc.subcore_barrier()

    pltpu.sync_copy(deg_sh, dis)
    @plsc.parallel_loop(0, N // 16, unroll=4)
    def _(i):
        v = dis[pl.ds(i * 16, 16)]
        ok = v > 0.0
        y = _rsqrt16(jnp.where(ok, v, 1.0))
        dis[pl.ds(i * 16, 16)] = jnp.where(ok, y, 0.0)

    _zero_rows(rowA, B)
    _zero_acc_slice(acc, rowA, sid)
    plsc.subcore_barrier()

    def group(g, _):
        r0 = wid * CPT + g * GC
        pltpu.sync_copy(src2_h.at[pl.ds(r0, GC)], srcv)
        pltpu.sync_copy(dst2_h.at[pl.ds(r0, GC)], dstv)
        pltpu.sync_copy(w2_h.at[pl.ds(r0, GC)], wv)
        @plsc.parallel_loop(0, GC, unroll=2)
        def _(j):
            for k in range(B // 16):
                s = srcv[j, pl.ds(k * 16, 16)]
                d = dstv[j, pl.ds(k * 16, 16)]
                wq = wv[j, pl.ds(k * 16, 16)]
                a = plsc.load_gather(dis, [s])
                b = plsc.load_gather(dis, [d])
                normv[pl.ds(j * B + k * 16, 16)] = -(a * wq * b)
        pltpu.sync_copy(normv, norm_h.at[wid, pl.ds(g * GE, GE)])
        _prop_group(x_h, srcv, dstv, normv, rowA, rowB,
                    semA, semB, ssemA, ssemB, acc)
        return 0
    lax.fori_loop(0, NG, group, 0)
    plsc.subcore_barrier()
    _writeout(acc, rowA, tx1p_h, cid, sid)


def _sc_pass2_body(src2_h, dst2_h, norm_h, tx1_h, tx2p_h,
                   srcv, dstv, normv, rowA, rowB,
                   semA, semB, ssemA, ssemB, acc):
    cid = lax.axis_index("c")
    sid = lax.axis_index("s")
    wid = cid * NS + sid

    _zero_rows(rowA, B)
    _zero_acc_slice(acc, rowA, sid)
    plsc.subcore_barrier()

    def group(g, _):
        r0 = wid * CPT + g * GC
        pltpu.sync_copy(src2_h.at[pl.ds(r0, GC)], srcv)
        pltpu.sync_copy(dst2_h.at[pl.ds(r0, GC)], dstv)
        pltpu.sync_copy(norm_h.at[wid, pl.ds(g * GE, GE)], normv)
        _prop_group(tx1_h, srcv, dstv, normv, rowA, rowB,
                    semA, semB, ssemA, ssemB, acc)
        return 0
    lax.fori_loop(0, NG, group, 0)
    plsc.subcore_barrier()
    _writeout(acc, rowA, tx2p_h, cid, sid)


@functools.lru_cache(maxsize=1)
def _sc_kernels():
    mesh = plsc.VectorSubcoreMesh(
        core_axis_name="c", subcore_axis_name="s",
        num_cores=NC, num_subcores=NS)
    cp = pltpu.CompilerParams(use_tc_tiling_on_sc=False,
                              needs_layout_passes=False)
    sc1 = pl.kernel(
        _sc_pass1_body,
        out_type=(jax.ShapeDtypeStruct((NC, N, F), jnp.float32),
                  jax.ShapeDtypeStruct((NC * NS, EPT), jnp.float32)),
        mesh=mesh,
        compiler_params=cp,
        scratch_types=[
            pltpu.VMEM((GC, B), jnp.int32),
            pltpu.VMEM((GC, B), jnp.int32),
            pltpu.VMEM((GC, B), jnp.float32),
            pltpu.VMEM((GE,), jnp.float32),
            pltpu.VMEM((N,), jnp.float32),
            pltpu.VMEM((B, F), jnp.float32),
            pltpu.VMEM((B, F), jnp.float32),
            pltpu.VMEM((DGC, DB), jnp.int32),
            pltpu.VMEM((DGC, DB), jnp.float32),
            pltpu.SemaphoreType.DMA,
            pltpu.SemaphoreType.DMA,
            pltpu.SemaphoreType.DMA,
            pltpu.SemaphoreType.DMA,
            pltpu.VMEM_SHARED((N,), jnp.float32),
            pltpu.VMEM_SHARED((N, F), jnp.float32),
        ],
    )
    sc2 = pl.kernel(
        _sc_pass2_body,
        out_type=jax.ShapeDtypeStruct((NC, N, F), jnp.float32),
        mesh=mesh,
        compiler_params=cp,
        scratch_types=[
            pltpu.VMEM((GC, B), jnp.int32),
            pltpu.VMEM((GC, B), jnp.int32),
            pltpu.VMEM((GE,), jnp.float32),
            pltpu.VMEM((B, F), jnp.float32),
            pltpu.VMEM((B, F), jnp.float32),
            pltpu.SemaphoreType.DMA,
            pltpu.SemaphoreType.DMA,
            pltpu.SemaphoreType.DMA,
            pltpu.SemaphoreType.DMA,
            pltpu.VMEM_SHARED((N, F), jnp.float32),
        ],
    )
    return sc1, sc2


def _combine_body(a_ref, b_ref, o_ref):
    o_ref[...] = a_ref[...] + b_ref[...]


def _dense_body(x_ref, t1_ref, t2a_ref, t2b_ref, w_ref, bias_ref, wco_ref,
                o_ref):
    x = x_ref[...]
    t1 = t1_ref[...]
    t2 = 2.0 * (t2a_ref[...] + t2b_ref[...]) - x
    w = w_ref[...]
    g = (jnp.dot(x, w[0:F, :], preferred_element_type=jnp.float32)
         + jnp.dot(t1, w[F:2 * F, :], preferred_element_type=jnp.float32)
         + jnp.dot(t2, w[2 * F:3 * F, :], preferred_element_type=jnp.float32)
         + bias_ref[...])
    i_g = jax.nn.sigmoid(g[:, 0:F])
    t_g = jnp.tanh(g[:, F:2 * F])
    c = i_g * t_g
    o_g = jax.nn.sigmoid(g[:, 2 * F:3 * F] + wco_ref[...] * c)
    h = o_g * jnp.tanh(c)
    o_ref[...] = jnp.maximum(h, 0.0)


_RB = 1000
_GRID = N // _RB

_combine = pl.pallas_call(
    _combine_body,
    grid=(_GRID,),
    in_specs=[pl.BlockSpec((_RB, F), lambda i: (i, 0))] * 2,
    out_specs=pl.BlockSpec((_RB, F), lambda i: (i, 0)),
    out_shape=jax.ShapeDtypeStruct((N, F), jnp.float32),
)

_dense = pl.pallas_call(
    _dense_body,
    grid=(_GRID,),
    in_specs=[
        pl.BlockSpec((_RB, F), lambda i: (i, 0)),
        pl.BlockSpec((_RB, F), lambda i: (i, 0)),
        pl.BlockSpec((_RB, F), lambda i: (i, 0)),
        pl.BlockSpec((_RB, F), lambda i: (i, 0)),
        pl.BlockSpec((3 * F, 3 * F), lambda i: (0, 0)),
        pl.BlockSpec((1, 3 * F), lambda i: (0, 0)),
        pl.BlockSpec((1, F), lambda i: (0, 0)),
    ],
    out_specs=pl.BlockSpec((_RB, F), lambda i: (i, 0)),
    out_shape=jax.ShapeDtypeStruct((N, F), jnp.float32),
)


def kernel(edge_index_list, node_feats_list, edge_feats_list,
           nodes_mask_list, params):
    ei = edge_index_list[-1].astype(jnp.int32)
    src2 = ei[0].reshape(ROWS2D, B)
    dst2 = ei[1].reshape(ROWS2D, B)
    w = edge_feats_list[-1].astype(jnp.float32)
    w2 = w.reshape(ROWS2D, B)
    srcd = ei[0].reshape(DROWS, DB)
    wd = w.reshape(DROWS, DB)
    x = node_feats_list[-1].astype(jnp.float32)

    sc1, sc2 = _sc_kernels()
    tx1p, norm2 = sc1(src2, dst2, w2, srcd, wd, x)
    tx1 = _combine(tx1p[0], tx1p[1])
    tx2p = sc2(src2, dst2, norm2, tx1)

    gates = "ico"
    wcat = jnp.concatenate(
        [jnp.concatenate([params["W_x_" + g][k] for g in gates], axis=1)
         for k in range(3)], axis=0)
    bias = jnp.concatenate(
        [params["b_x_" + g] + params["b_h_" + g] + params["b_" + g][0]
         for g in gates])[None, :]
    return _dense(x, tx1, tx2p[0], tx2p[1], wcat, bias, params["w_c_o"])
